# Initial kernel scaffold; baseline (speedup 1.0000x reference)
import functools
import math

import jax
import jax.numpy as jnp
from jax import lax
from jax.experimental import pallas as pl
from jax.experimental.pallas import tpu as pltpu

B = 2
N = 2048
DIM = 1024
HEADS = 16
DIM_HEAD = 64
G = 2
W = 256
NQ = 128
NKV = 256
INNER = HEADS * DIM_HEAD
NB = N // W
NWIN = NB - 1
BW = B * NWIN
CTX = N - W
SCALE = DIM_HEAD ** -0.5
LOGPAD = 128
F32 = jnp.float32


def _qkv_body(x_ref, g_ref, b_ref, wqkv_ref, wr_ref, qkv_ref, log_ref):
    xb = x_ref[0]
    mu = jnp.mean(xb, axis=1, keepdims=True)
    var = jnp.mean((xb - mu) ** 2, axis=1, keepdims=True)
    xn = (xb - mu) / jnp.sqrt(var + 1e-5) * g_ref[0][None, :] + b_ref[0][None, :]
    qkv_ref[0] = jnp.dot(xn, wqkv_ref[...], preferred_element_type=F32)
    log_ref[0] = jnp.dot(xb, wr_ref[...], preferred_element_type=F32)


def _qkv_call(x3, ln_g, ln_b, Wqkv, Wr_pad):
    ncol = 3
    return pl.pallas_call(
        _qkv_body,
        grid=(ncol, B * NB),
        in_specs=[
            pl.BlockSpec((1, W, DIM), lambda c, t: (t, 0, 0)),
            pl.BlockSpec((1, DIM), lambda c, t: (0, 0)),
            pl.BlockSpec((1, DIM), lambda c, t: (0, 0)),
            pl.BlockSpec((DIM, INNER), lambda c, t: (0, c)),
            pl.BlockSpec((DIM, LOGPAD), lambda c, t: (0, 0)),
        ],
        out_specs=[
            pl.BlockSpec((1, W, INNER), lambda c, t: (t, 0, c)),
            pl.BlockSpec((1, W, LOGPAD), lambda c, t: (t, 0, 0)),
        ],
        out_shape=[
            jax.ShapeDtypeStruct((B * NB, W, 3 * INNER), F32),
            jax.ShapeDtypeStruct((B * NB, W, LOGPAD), F32),
        ],
    )(x3, ln_g, ln_b, Wqkv, Wr_pad)


def _local_body(q_ref, kc_ref, vc_ref, kp_ref, vp_ref, wo_ref, out_ref, o_scr):
    t = pl.program_id(0)
    i = t % NB
    first = i == 0
    rows = lax.broadcasted_iota(jnp.int32, (W, W), 0)
    cols = lax.broadcasted_iota(jnp.int32, (W, W), 1)
    causal = rows >= cols
    neg = jnp.full((W, W), -1e9, F32)
    for h in range(HEADS):
        sl = slice(h * DIM_HEAD, (h + 1) * DIM_HEAD)
        qh = q_ref[0, :, sl]
        kc = kc_ref[0, :, sl]
        vc = vc_ref[0, :, sl]
        kp = kp_ref[0, :, sl]
        vp = vp_ref[0, :, sl]
        dn = (((1,), (1,)), ((), ()))
        sim_c = lax.dot_general(qh, kc, dn, preferred_element_type=F32) * SCALE
        sim_p = lax.dot_general(qh, kp, dn, preferred_element_type=F32) * SCALE
        sim_c = jnp.where(causal, sim_c, -1e9)
        sim_p = jnp.where(first, neg, sim_p)
        m = jnp.maximum(jnp.max(sim_c, axis=1, keepdims=True),
                        jnp.max(sim_p, axis=1, keepdims=True))
        pc = jnp.exp(sim_c - m)
        pp = jnp.exp(sim_p - m)
        den = jnp.sum(pc, axis=1, keepdims=True) + jnp.sum(pp, axis=1, keepdims=True)
        o = (jnp.dot(pc, vc, preferred_element_type=F32)
             + jnp.dot(pp, vp, preferred_element_type=F32)) / den
        o_scr[:, sl] = o
    out_ref[0] = jnp.dot(o_scr[...], wo_ref[...], preferred_element_type=F32)


def _local_call(qkv, Wo_local):
    def prev_idx(t):
        return jnp.where(t % NB == 0, t, t - 1)

    return pl.pallas_call(
        _local_body,
        grid=(B * NB,),
        in_specs=[
            pl.BlockSpec((1, W, INNER), lambda t: (t, 0, 0)),
            pl.BlockSpec((1, W, INNER), lambda t: (t, 0, 1)),
            pl.BlockSpec((1, W, INNER), lambda t: (t, 0, 2)),
            pl.BlockSpec((1, W, INNER), lambda t: (prev_idx(t), 0, 1)),
            pl.BlockSpec((1, W, INNER), lambda t: (prev_idx(t), 0, 2)),
            pl.BlockSpec((INNER, DIM), lambda t: (0, 0)),
        ],
        out_specs=pl.BlockSpec((1, W, DIM), lambda t: (t, 0, 0)),
        out_shape=jax.ShapeDtypeStruct((B * NB, W, DIM), F32),
        scratch_shapes=[pltpu.VMEM((W, INNER), F32)],
    )(qkv, qkv, qkv, qkv, qkv, Wo_local)


def _grouped_body(xq_ref, xc_ref, qs_ref, kvs_ref, kvb_ref, gq_ref, gc_ref,
                  wq_ref, wkv_ref, wo_ref, nk_ref, nv_ref, out_ref, o_scr):
    dn = (((1,), (1,)), ((), ()))
    xq = xq_ref[0]
    nq = jnp.sqrt(jnp.sum(xq * xq, axis=1, keepdims=True))
    xqn = xq / jnp.maximum(nq, 1e-12) * (DIM ** 0.5) * gq_ref[0, 0][None, :]
    xc = xc_ref[0]
    nc = jnp.sqrt(jnp.sum(xc * xc, axis=1, keepdims=True))
    xcn = xc / jnp.maximum(nc, 1e-12) * (DIM ** 0.5) * gc_ref[0, 0][None, :]
    q = lax.dot_general(xqn, wq_ref[0], dn, preferred_element_type=F32)
    kv = lax.dot_general(xcn, wkv_ref[0], dn, preferred_element_type=F32)
    k = kv[:, :INNER]
    v = kv[:, INNER:] * kvs_ref[0, 0][:, None]
    bias = kvb_ref[0, 0][None, :]
    for h in range(HEADS):
        sl = slice(h * DIM_HEAD, (h + 1) * DIM_HEAD)
        qh = q[:, sl]
        kh = k[:, sl]
        vh = v[:, sl]
        sim = lax.dot_general(qh, kh, dn, preferred_element_type=F32) * SCALE + bias
        nkh = nk_ref[0, h][None, :]
        nvh = nv_ref[0, h][None, :]
        simn = jnp.sum(qh * nkh, axis=1) * SCALE
        m = jnp.maximum(jnp.max(sim, axis=1), simn)
        p = jnp.exp(sim - m[:, None])
        pn = jnp.exp(simn - m)
        den = jnp.sum(p, axis=1) + pn
        o = (jnp.dot(p, vh, preferred_element_type=F32) + pn[:, None] * nvh)
        o_scr[:, sl] = o / den[:, None]
    out = lax.dot_general(o_scr[...], wo_ref[0], dn, preferred_element_type=F32)
    out_ref[0] = out * qs_ref[0, 0][:, None]


def _grouped_call(xq_r, xc_r, qs, kvs, kvb, gq, gc, Wq, Wkv, Wout, nk, nv):
    return pl.pallas_call(
        _grouped_body,
        grid=(G, BW),
        in_specs=[
            pl.BlockSpec((1, NQ, DIM), lambda g, w: (w * G + g, 0, 0)),
            pl.BlockSpec((1, NKV, DIM), lambda g, w: (w * G + g, 0, 0)),
            pl.BlockSpec((1, 1, NQ), lambda g, w: (w * G + g, 0, 0)),
            pl.BlockSpec((1, 1, NKV), lambda g, w: (w * G + g, 0, 0)),
            pl.BlockSpec((1, 1, NKV), lambda g, w: (w * G + g, 0, 0)),
            pl.BlockSpec((1, 1, DIM), lambda g, w: (g, 0, 0)),
            pl.BlockSpec((1, 1, DIM), lambda g, w: (g, 0, 0)),
            pl.BlockSpec((1, INNER, DIM), lambda g, w: (g, 0, 0)),
            pl.BlockSpec((1, 2 * INNER, DIM), lambda g, w: (g, 0, 0)),
            pl.BlockSpec((1, DIM, INNER), lambda g, w: (g, 0, 0)),
            pl.BlockSpec((1, HEADS, DIM_HEAD), lambda g, w: (g, 0, 0)),
            pl.BlockSpec((1, HEADS, DIM_HEAD), lambda g, w: (g, 0, 0)),
        ],
        out_specs=pl.BlockSpec((1, NQ, DIM), lambda g, w: (w * G + g, 0, 0)),
        out_shape=jax.ShapeDtypeStruct((BW * G, NQ, DIM), F32),
        scratch_shapes=[pltpu.VMEM((NQ, INNER), F32)],
    )(xq_r, xc_r, qs, kvs, kvb, gq, gc, Wq, Wkv, Wout, nk, nv)


def kernel(x, ln_g, ln_b, Wqkv, Wo_local, Wr_q, Wr_kv, gamma_q, gamma_c,
           Wq, Wkv, Wout, null_kv):
    x3 = x.reshape(B * NB, W, DIM)
    Wr_pad = jnp.concatenate(
        [Wr_q, Wr_kv, jnp.zeros((DIM, LOGPAD - 2 * G), F32)], axis=1)

    qkv, logits = _qkv_call(x3, ln_g.reshape(1, DIM), ln_b.reshape(1, DIM),
                            Wqkv, Wr_pad)
    local_out = _local_call(qkv, Wo_local).reshape(B, N, DIM)

    lq = logits[:, :, 0:G].reshape(B, NB, W, G)[:, 1:]
    lq = lq.transpose(0, 1, 3, 2).reshape(BW, G, W)
    q_vals, q_idx = lax.top_k(lq, NQ)
    q_scores = jax.nn.sigmoid(q_vals)

    lkv = logits[:, :, G:2 * G].reshape(B, N, G)[:, :CTX]
    lkv = lkv.transpose(0, 2, 1)
    win = jnp.arange(NWIN)
    limits = (win + 1) * W
    ctx_mask = jnp.arange(CTX)[None, :] < limits[:, None]
    lkv_m = jnp.where(ctx_mask[None, :, None, :], lkv[:, None, :, :], -1e9)
    lkv_m = lkv_m.reshape(BW, G, CTX)
    kv_vals, kv_idx = lax.top_k(lkv_m, NKV)
    kv_scores = jax.nn.sigmoid(kv_vals)
    lim_bw = limits[jnp.arange(BW) % NWIN]
    kv_valid = kv_idx < lim_bw[:, None, None]
    kv_bias = jnp.where(kv_valid, 0.0, -1e9).astype(F32)

    b_of = jnp.arange(BW) // NWIN
    w_of = jnp.arange(BW) % NWIN
    xf = x.reshape(B * N, DIM)
    q_glob = q_idx + ((w_of + 1) * W)[:, None, None] + (b_of * N)[:, None, None]
    kv_glob = kv_idx + (b_of * N)[:, None, None]
    xq_r = jnp.take(xf, q_glob.reshape(-1), axis=0).reshape(BW * G, NQ, DIM)
    xc_r = jnp.take(xf, kv_glob.reshape(-1), axis=0).reshape(BW * G, NKV, DIM)

    nk = null_kv[0, :, :, 0, :]
    nv = null_kv[1, :, :, 0, :]
    attn_out = _grouped_call(
        xq_r, xc_r,
        q_scores.reshape(BW * G, 1, NQ),
        kv_scores.reshape(BW * G, 1, NKV),
        kv_bias.reshape(BW * G, 1, NKV),
        gamma_q.reshape(G, 1, DIM), gamma_c.reshape(G, 1, DIM),
        Wq, Wkv, Wout, nk, nv)

    af = attn_out.reshape(BW, G * NQ, DIM)
    qif = q_idx.reshape(BW, G * NQ)
    rows = jnp.arange(BW)[:, None]
    out = jnp.zeros((BW, W, DIM), F32).at[rows, qif].add(af)
    counts = jnp.zeros((BW, W), F32).at[rows, qif].add(1.0)
    out = out / jnp.maximum(counts[..., None], 1e-5)
    out = out.reshape(B, NWIN * W, DIM)
    out = jnp.pad(out, ((0, 0), (W, 0), (0, 0)))
    return out + local_out


# speedup vs baseline: 2.1978x; 2.1978x over previous
import functools
import math

import jax
import jax.numpy as jnp
from jax import lax
from jax.experimental import pallas as pl
from jax.experimental.pallas import tpu as pltpu
from jax.experimental.pallas import tpu_sc as plsc

B = 2
N = 2048
DIM = 1024
HEADS = 16
DIM_HEAD = 64
G = 2
W = 256
NQ = 128
NKV = 256
INNER = HEADS * DIM_HEAD
NB = N // W
NWIN = NB - 1
BW = B * NWIN
CTX = N - W
SCALE = DIM_HEAD ** -0.5
LOGPAD = 128
F32 = jnp.float32


def _qkv_body(x_ref, g_ref, b_ref, wqkv_ref, wr_ref, qkv_ref, log_ref):
    xb = x_ref[0]
    mu = jnp.mean(xb, axis=1, keepdims=True)
    var = jnp.mean((xb - mu) ** 2, axis=1, keepdims=True)
    xn = (xb - mu) / jnp.sqrt(var + 1e-5) * g_ref[0][None, :] + b_ref[0][None, :]
    qkv_ref[0] = jnp.dot(xn, wqkv_ref[...], preferred_element_type=F32)
    log_ref[0, 0] = jnp.dot(xb, wr_ref[...], preferred_element_type=F32)


def _qkv_call(x3, ln_g, ln_b, Wqkv, Wr_pad):
    ncol = 3
    return pl.pallas_call(
        _qkv_body,
        grid=(ncol, B * NB),
        in_specs=[
            pl.BlockSpec((1, W, DIM), lambda c, t: (t, 0, 0)),
            pl.BlockSpec((1, DIM), lambda c, t: (0, 0)),
            pl.BlockSpec((1, DIM), lambda c, t: (0, 0)),
            pl.BlockSpec((DIM, INNER), lambda c, t: (0, c)),
            pl.BlockSpec((DIM, LOGPAD), lambda c, t: (0, 0)),
        ],
        out_specs=[
            pl.BlockSpec((1, W, INNER), lambda c, t: (t, 0, c)),
            pl.BlockSpec((1, 1, W, LOGPAD), lambda c, t: (t, c, 0, 0)),
        ],
        out_shape=[
            jax.ShapeDtypeStruct((B * NB, W, 3 * INNER), F32),
            jax.ShapeDtypeStruct((B * NB, 3, W, LOGPAD), F32),
        ],
    )(x3, ln_g, ln_b, Wqkv, Wr_pad)


def _local_body(q_ref, kc_ref, vc_ref, kp_ref, vp_ref, wo_ref, out_ref, o_scr):
    t = pl.program_id(0)
    i = t % NB
    first = i == 0
    rows = lax.broadcasted_iota(jnp.int32, (W, W), 0)
    cols = lax.broadcasted_iota(jnp.int32, (W, W), 1)
    causal = rows >= cols
    neg = jnp.full((W, W), -1e9, F32)
    for h in range(HEADS):
        sl = slice(h * DIM_HEAD, (h + 1) * DIM_HEAD)
        qh = q_ref[0, :, sl]
        kc = kc_ref[0, :, sl]
        vc = vc_ref[0, :, sl]
        kp = kp_ref[0, :, sl]
        vp = vp_ref[0, :, sl]
        dn = (((1,), (1,)), ((), ()))
        sim_c = lax.dot_general(qh, kc, dn, preferred_element_type=F32) * SCALE
        sim_p = lax.dot_general(qh, kp, dn, preferred_element_type=F32) * SCALE
        sim_c = jnp.where(causal, sim_c, -1e9)
        sim_p = jnp.where(first, neg, sim_p)
        m = jnp.maximum(jnp.max(sim_c, axis=1, keepdims=True),
                        jnp.max(sim_p, axis=1, keepdims=True))
        pc = jnp.exp(sim_c - m)
        pp = jnp.exp(sim_p - m)
        den = jnp.sum(pc, axis=1, keepdims=True) + jnp.sum(pp, axis=1, keepdims=True)
        o = (jnp.dot(pc, vc, preferred_element_type=F32)
             + jnp.dot(pp, vp, preferred_element_type=F32)) / den
        o_scr[:, sl] = o
    out_ref[0] = jnp.dot(o_scr[...], wo_ref[...], preferred_element_type=F32)


def _local_call(qkv, Wo_local):
    def prev_idx(t):
        return jnp.where(t % NB == 0, t, t - 1)

    return pl.pallas_call(
        _local_body,
        grid=(B * NB,),
        in_specs=[
            pl.BlockSpec((1, W, INNER), lambda t: (t, 0, 0)),
            pl.BlockSpec((1, W, INNER), lambda t: (t, 0, 1)),
            pl.BlockSpec((1, W, INNER), lambda t: (t, 0, 2)),
            pl.BlockSpec((1, W, INNER), lambda t: (prev_idx(t), 0, 1)),
            pl.BlockSpec((1, W, INNER), lambda t: (prev_idx(t), 0, 2)),
            pl.BlockSpec((INNER, DIM), lambda t: (0, 0)),
        ],
        out_specs=pl.BlockSpec((1, W, DIM), lambda t: (t, 0, 0)),
        out_shape=jax.ShapeDtypeStruct((B * NB, W, DIM), F32),
        scratch_shapes=[pltpu.VMEM((W, INNER), F32)],
    )(qkv, qkv, qkv, qkv, qkv, Wo_local)


def _grouped_body(xq_ref, xc_ref, qs_ref, kvs_ref, kvb_ref, gq_ref, gc_ref,
                  wq_ref, wkv_ref, wo_ref, nk_ref, nv_ref, out_ref, o_scr):
    dn = (((1,), (1,)), ((), ()))
    xq = xq_ref[...]
    nq = jnp.sqrt(jnp.sum(xq * xq, axis=1, keepdims=True))
    xqn = xq / jnp.maximum(nq, 1e-12) * (DIM ** 0.5) * gq_ref[0, 0][None, :]
    xc = xc_ref[...]
    nc = jnp.sqrt(jnp.sum(xc * xc, axis=1, keepdims=True))
    xcn = xc / jnp.maximum(nc, 1e-12) * (DIM ** 0.5) * gc_ref[0, 0][None, :]
    q = lax.dot_general(xqn, wq_ref[0], dn, preferred_element_type=F32)
    kv = lax.dot_general(xcn, wkv_ref[0], dn, preferred_element_type=F32)
    k = kv[:, :INNER]
    v = kv[:, INNER:] * kvs_ref[0, 0][:, None]
    bias = kvb_ref[0, 0][None, :]
    for h in range(HEADS):
        sl = slice(h * DIM_HEAD, (h + 1) * DIM_HEAD)
        qh = q[:, sl]
        kh = k[:, sl]
        vh = v[:, sl]
        sim = lax.dot_general(qh, kh, dn, preferred_element_type=F32) * SCALE + bias
        nkh = nk_ref[0, h][None, :]
        nvh = nv_ref[0, h][None, :]
        simn = jnp.sum(qh * nkh, axis=1) * SCALE
        m = jnp.maximum(jnp.max(sim, axis=1), simn)
        p = jnp.exp(sim - m[:, None])
        pn = jnp.exp(simn - m)
        den = jnp.sum(p, axis=1) + pn
        o = (jnp.dot(p, vh, preferred_element_type=F32) + pn[:, None] * nvh)
        o_scr[:, sl] = o / den[:, None]
    out = lax.dot_general(o_scr[...], wo_ref[0], dn, preferred_element_type=F32)
    out_ref[0] = out * qs_ref[0, 0][:, None]


def _grouped_call(gat, qs, kvs, kvb, gq, gc, Wq, Wkv, Wout, nk, nv):
    qrows = BW * G * NQ // NKV
    return pl.pallas_call(
        _grouped_body,
        grid=(G, BW),
        in_specs=[
            pl.BlockSpec((NQ, DIM), lambda g, w: (w * G + g, 0)),
            pl.BlockSpec((NKV, DIM), lambda g, w: (qrows + w * G + g, 0)),
            pl.BlockSpec((1, 1, NQ), lambda g, w: (w * G + g, 0, 0)),
            pl.BlockSpec((1, 1, NKV), lambda g, w: (w * G + g, 0, 0)),
            pl.BlockSpec((1, 1, NKV), lambda g, w: (w * G + g, 0, 0)),
            pl.BlockSpec((1, 1, DIM), lambda g, w: (g, 0, 0)),
            pl.BlockSpec((1, 1, DIM), lambda g, w: (g, 0, 0)),
            pl.BlockSpec((1, INNER, DIM), lambda g, w: (g, 0, 0)),
            pl.BlockSpec((1, 2 * INNER, DIM), lambda g, w: (g, 0, 0)),
            pl.BlockSpec((1, DIM, INNER), lambda g, w: (g, 0, 0)),
            pl.BlockSpec((1, HEADS, DIM_HEAD), lambda g, w: (g, 0, 0)),
            pl.BlockSpec((1, HEADS, DIM_HEAD), lambda g, w: (g, 0, 0)),
        ],
        out_specs=pl.BlockSpec((1, NQ, DIM), lambda g, w: (w * G + g, 0, 0)),
        out_shape=jax.ShapeDtypeStruct((BW * G, NQ, DIM), F32),
        scratch_shapes=[pltpu.VMEM((NQ, INNER), F32)],
    )(gat, gat, qs, kvs, kvb, gq, gc, Wq, Wkv, Wout, nk, nv)


NROWS = BW * G * (NQ + NKV)
NWORK = 32
PER_WORK = NROWS // NWORK
SC_CHUNK = 48
SC_NCH = PER_WORK // SC_CHUNK


def _sc_gather(xf, idx):
    mesh = plsc.VectorSubcoreMesh(core_axis_name="c", subcore_axis_name="s")

    @functools.partial(
        pl.kernel, mesh=mesh,
        out_type=jax.ShapeDtypeStruct((NROWS, DIM), F32),
        scratch_types=[
            pltpu.VMEM((SC_CHUNK,), jnp.int32),
            pltpu.VMEM((SC_CHUNK, DIM), F32),
            pltpu.SemaphoreType.DMA,
        ],
    )
    def gk(idx_hbm, xf_hbm, out_hbm, idx_v, rows_v, sem):
        wid = lax.axis_index("s") * 2 + lax.axis_index("c")
        base = wid * PER_WORK
        for ci in range(SC_NCH):
            off = base + ci * SC_CHUNK
            pltpu.sync_copy(idx_hbm.at[pl.ds(off, SC_CHUNK)], idx_v)
            pltpu.async_copy(xf_hbm.at[idx_v], rows_v, sem).wait()
            pltpu.sync_copy(rows_v, out_hbm.at[pl.ds(off, SC_CHUNK)])

    return gk(idx, xf)


def _scatter_body(af_ref, qif_ref, local_ref, out_ref):
    t = pl.program_id(0)
    routed = t % NB != 0
    loc = local_ref[0]
    qv = qif_ref[0, 0]
    pos = lax.broadcasted_iota(jnp.int32, (G * NQ, W), 1)
    P = (pos == qv[:, None]).astype(F32)
    af = af_ref[0]
    scat = lax.dot_general(P, af, (((0,), (0,)), ((), ())),
                           preferred_element_type=F32)
    cnt = jnp.sum(P, axis=0)
    add = scat / jnp.maximum(cnt, 1e-5)[:, None]
    out_ref[0] = loc + jnp.where(routed, add, 0.0)


def _scatter_call(af, qif, local3):
    def w_idx(t):
        return jnp.where(t % NB == 0, 0, t - t // NB - 1)

    return pl.pallas_call(
        _scatter_body,
        grid=(B * NB,),
        in_specs=[
            pl.BlockSpec((1, G * NQ, DIM), lambda t: (w_idx(t), 0, 0)),
            pl.BlockSpec((1, 1, G * NQ), lambda t: (w_idx(t), 0, 0)),
            pl.BlockSpec((1, W, DIM), lambda t: (t, 0, 0)),
        ],
        out_specs=pl.BlockSpec((1, W, DIM), lambda t: (t, 0, 0)),
        out_shape=jax.ShapeDtypeStruct((B * NB, W, DIM), F32),
    )(af, qif, local3)


def kernel(x, ln_g, ln_b, Wqkv, Wo_local, Wr_q, Wr_kv, gamma_q, gamma_c,
           Wq, Wkv, Wout, null_kv):
    x3 = x.reshape(B * NB, W, DIM)
    Wr_pad = jnp.concatenate(
        [Wr_q, Wr_kv, jnp.zeros((DIM, LOGPAD - 2 * G), F32)], axis=1)

    qkv, logits3 = _qkv_call(x3, ln_g.reshape(1, DIM), ln_b.reshape(1, DIM),
                             Wqkv, Wr_pad)
    logits = logits3[:, 0]
    local3 = _local_call(qkv, Wo_local)

    lq = logits[:, :, 0:G].reshape(B, NB, W, G)[:, 1:]
    lq = lq.transpose(0, 1, 3, 2).reshape(BW, G, W)
    q_vals, q_idx = lax.top_k(lq, NQ)
    q_scores = jax.nn.sigmoid(q_vals)

    lkv = logits[:, :, G:2 * G].reshape(B, N, G)[:, :CTX]
    lkv = lkv.transpose(0, 2, 1)
    win = jnp.arange(NWIN)
    limits = (win + 1) * W
    ctx_mask = jnp.arange(CTX)[None, :] < limits[:, None]
    lkv_m = jnp.where(ctx_mask[None, :, None, :], lkv[:, None, :, :], -1e9)
    lkv_m = lkv_m.reshape(BW, G, CTX)
    kv_vals, kv_idx = lax.top_k(lkv_m, NKV)
    kv_scores = jax.nn.sigmoid(kv_vals)
    lim_bw = limits[jnp.arange(BW) % NWIN]
    kv_valid = kv_idx < lim_bw[:, None, None]
    kv_bias = jnp.where(kv_valid, 0.0, -1e9).astype(F32)

    b_of = jnp.arange(BW) // NWIN
    w_of = jnp.arange(BW) % NWIN
    xf = x.reshape(B * N, DIM)
    q_glob = q_idx + ((w_of + 1) * W)[:, None, None] + (b_of * N)[:, None, None]
    kv_glob = kv_idx + (b_of * N)[:, None, None]
    all_idx = jnp.concatenate(
        [q_glob.reshape(-1), kv_glob.reshape(-1)]).astype(jnp.int32)
    gat = _sc_gather(xf, all_idx)

    nk = null_kv[0, :, :, 0, :]
    nv = null_kv[1, :, :, 0, :]
    attn_out = _grouped_call(
        gat,
        q_scores.reshape(BW * G, 1, NQ),
        kv_scores.reshape(BW * G, 1, NKV),
        kv_bias.reshape(BW * G, 1, NKV),
        gamma_q.reshape(G, 1, DIM), gamma_c.reshape(G, 1, DIM),
        Wq, Wkv, Wout, nk, nv)

    af = attn_out.reshape(BW, G * NQ, DIM)
    qif = q_idx.reshape(BW, 1, G * NQ)
    out = _scatter_call(af, qif, local3)
    return out.reshape(B, N, DIM)


# speedup vs baseline: 2.7617x; 1.2566x over previous
import functools
import math

import jax
import jax.numpy as jnp
from jax import lax
from jax.experimental import pallas as pl
from jax.experimental.pallas import tpu as pltpu
from jax.experimental.pallas import tpu_sc as plsc

B = 2
N = 2048
DIM = 1024
HEADS = 16
DIM_HEAD = 64
G = 2
W = 256
NQ = 128
NKV = 256
INNER = HEADS * DIM_HEAD
NB = N // W
NWIN = NB - 1
BW = B * NWIN
CTX = N - W
SCALE = DIM_HEAD ** -0.5
LOGPAD = 128
F32 = jnp.float32
BF16 = jnp.bfloat16


def _fused_body(x_ref, g_ref, b_ref, wqkv_ref, wr_ref, wo_ref,
                local_ref, log_ref, ring, o_scr):
    t = pl.program_id(0)
    xb = x_ref[0]
    mu = jnp.mean(xb, axis=1, keepdims=True)
    var = jnp.mean((xb - mu) ** 2, axis=1, keepdims=True)
    xn = (xb - mu) / jnp.sqrt(var + 1e-5) * g_ref[0][None, :] + b_ref[0][None, :]
    cur = t % 2
    ring[cur] = jnp.dot(xn.astype(BF16), wqkv_ref[...],
                        preferred_element_type=F32)
    log_ref[0] = jnp.dot(xb, wr_ref[...], preferred_element_type=F32)

    prv = 1 - cur
    first = t % NB == 0
    rows = lax.broadcasted_iota(jnp.int32, (W, W), 0)
    cols = lax.broadcasted_iota(jnp.int32, (W, W), 1)
    causal = rows >= cols
    neg = jnp.full((W, W), -1e9, F32)
    dn = (((1,), (1,)), ((), ()))
    for h in range(HEADS):
        sl = slice(h * DIM_HEAD, (h + 1) * DIM_HEAD)
        slk = slice(INNER + h * DIM_HEAD, INNER + (h + 1) * DIM_HEAD)
        slv = slice(2 * INNER + h * DIM_HEAD, 2 * INNER + (h + 1) * DIM_HEAD)
        qh = ring[cur, :, sl] * SCALE
        kc = ring[cur, :, slk]
        vc = ring[cur, :, slv]
        kp = ring[prv, :, slk]
        vp = jnp.where(first, jnp.zeros((W, DIM_HEAD), F32),
                       ring[prv, :, slv])
        sim_c = lax.dot_general(qh, kc, dn, preferred_element_type=F32)
        sim_p = lax.dot_general(qh, kp, dn, preferred_element_type=F32)
        sim_c = jnp.where(causal, sim_c, -1e9)
        sim_p = jnp.where(first, neg, sim_p)
        m = jnp.maximum(jnp.max(sim_c, axis=1, keepdims=True),
                        jnp.max(sim_p, axis=1, keepdims=True))
        pc = jnp.exp(sim_c - m)
        pp = jnp.exp(sim_p - m)
        den = jnp.sum(pc, axis=1, keepdims=True) + jnp.sum(pp, axis=1, keepdims=True)
        o = (jnp.dot(pc, vc, preferred_element_type=F32)
             + jnp.dot(pp, vp, preferred_element_type=F32)) / den
        o_scr[:, sl] = o.astype(BF16)
    local_ref[0] = jnp.dot(o_scr[...], wo_ref[...], preferred_element_type=F32)


def _fused_call(x3, ln_g, ln_b, Wqkv, Wr_pad, Wo_local):
    return pl.pallas_call(
        _fused_body,
        grid=(B * NB,),
        in_specs=[
            pl.BlockSpec((1, W, DIM), lambda t: (t, 0, 0)),
            pl.BlockSpec((1, DIM), lambda t: (0, 0)),
            pl.BlockSpec((1, DIM), lambda t: (0, 0)),
            pl.BlockSpec((DIM, 3 * INNER), lambda t: (0, 0)),
            pl.BlockSpec((DIM, LOGPAD), lambda t: (0, 0)),
            pl.BlockSpec((INNER, DIM), lambda t: (0, 0)),
        ],
        out_specs=[
            pl.BlockSpec((1, W, DIM), lambda t: (t, 0, 0)),
            pl.BlockSpec((1, W, LOGPAD), lambda t: (t, 0, 0)),
        ],
        out_shape=[
            jax.ShapeDtypeStruct((B * NB, W, DIM), F32),
            jax.ShapeDtypeStruct((B * NB, W, LOGPAD), F32),
        ],
        scratch_shapes=[pltpu.VMEM((2, W, 3 * INNER), F32),
                        pltpu.VMEM((W, INNER), BF16)],
    )(x3, ln_g, ln_b, Wqkv, Wr_pad, Wo_local)


def _grouped_body(xq_ref, xc_ref, qs_ref, kvs_ref, kvb_ref, gq_ref, gc_ref,
                  wq_ref, wkv_ref, wo_ref, nk_ref, nv_ref, out_ref, o_scr):
    dn = (((1,), (1,)), ((), ()))
    xq = xq_ref[...]
    nq = jnp.sqrt(jnp.sum(xq * xq, axis=1, keepdims=True))
    xqn = xq / jnp.maximum(nq, 1e-12) * (DIM ** 0.5) * gq_ref[0, 0][None, :]
    xc = xc_ref[...]
    nc = jnp.sqrt(jnp.sum(xc * xc, axis=1, keepdims=True))
    xcn = xc / jnp.maximum(nc, 1e-12) * (DIM ** 0.5) * gc_ref[0, 0][None, :]
    q = lax.dot_general(xqn.astype(BF16), wq_ref[0], dn,
                        preferred_element_type=F32)
    kv = lax.dot_general(xcn.astype(BF16), wkv_ref[0], dn,
                         preferred_element_type=F32)
    k = kv[:, :INNER]
    v = kv[:, INNER:] * kvs_ref[0, 0][:, None]
    bias = kvb_ref[0, 0][None, :]
    for h in range(HEADS):
        sl = slice(h * DIM_HEAD, (h + 1) * DIM_HEAD)
        qh = q[:, sl] * SCALE
        kh = k[:, sl]
        vh = v[:, sl]
        sim = lax.dot_general(qh, kh, dn, preferred_element_type=F32) + bias
        nkh = nk_ref[0, h][None, :]
        nvh = nv_ref[0, h][None, :]
        simn = jnp.sum(qh * nkh, axis=1)
        m = jnp.maximum(jnp.max(sim, axis=1), simn)
        p = jnp.exp(sim - m[:, None])
        pn = jnp.exp(simn - m)
        den = jnp.sum(p, axis=1) + pn
        o = (jnp.dot(p, vh, preferred_element_type=F32) + pn[:, None] * nvh)
        o_scr[:, sl] = (o / den[:, None]).astype(BF16)
    out = lax.dot_general(o_scr[...], wo_ref[0], dn, preferred_element_type=F32)
    out_ref[0] = out * qs_ref[0, 0][:, None]


def _grouped_call(gat, qs, kvs, kvb, gq, gc, Wq, Wkv, Wout, nk, nv):
    qrows = BW * G * NQ // NKV
    return pl.pallas_call(
        _grouped_body,
        grid=(G, BW),
        in_specs=[
            pl.BlockSpec((NQ, DIM), lambda g, w: (w * G + g, 0)),
            pl.BlockSpec((NKV, DIM), lambda g, w: (qrows + w * G + g, 0)),
            pl.BlockSpec((1, 1, NQ), lambda g, w: (w * G + g, 0, 0)),
            pl.BlockSpec((1, 1, NKV), lambda g, w: (w * G + g, 0, 0)),
            pl.BlockSpec((1, 1, NKV), lambda g, w: (w * G + g, 0, 0)),
            pl.BlockSpec((1, 1, DIM), lambda g, w: (g, 0, 0)),
            pl.BlockSpec((1, 1, DIM), lambda g, w: (g, 0, 0)),
            pl.BlockSpec((1, INNER, DIM), lambda g, w: (g, 0, 0)),
            pl.BlockSpec((1, 2 * INNER, DIM), lambda g, w: (g, 0, 0)),
            pl.BlockSpec((1, DIM, INNER), lambda g, w: (g, 0, 0)),
            pl.BlockSpec((1, HEADS, DIM_HEAD), lambda g, w: (g, 0, 0)),
            pl.BlockSpec((1, HEADS, DIM_HEAD), lambda g, w: (g, 0, 0)),
        ],
        out_specs=pl.BlockSpec((1, NQ, DIM), lambda g, w: (w * G + g, 0, 0)),
        out_shape=jax.ShapeDtypeStruct((BW * G, NQ, DIM), F32),
        scratch_shapes=[pltpu.VMEM((NQ, INNER), BF16)],
    )(gat, gat, qs, kvs, kvb, gq, gc, Wq, Wkv, Wout, nk, nv)


def _topk_body(R, L, K, v_ref, idx_ref, val_ref, ut_scr):
    _SIGN = jnp.int32(-2 ** 31)
    r_ = lax.broadcasted_iota(jnp.int32, (L, L), 0)
    c_ = lax.broadcasted_iota(jnp.int32, (L, L), 1)
    ut_scr[...] = (r_ < c_).astype(F32)

    v = v_ref[...]
    b = lax.bitcast_convert_type(v, jnp.int32)
    key = jnp.where(b >= 0, b, ~b ^ _SIGN)

    def step(i, t):
        cand = t | (jnp.int32(1) << (31 - i))
        cnt = jnp.sum((key >= (cand ^ _SIGN)).astype(jnp.int32),
                      axis=1, keepdims=True)
        return jnp.where(cnt >= K, cand, t)

    t = lax.fori_loop(0, 32, step, jnp.zeros((R, 1), jnp.int32))
    kth = t ^ _SIGN
    gt = key > kth
    eq = key == kth
    fill = (K - jnp.sum(gt.astype(jnp.int32), axis=1, keepdims=True)).astype(F32)
    eq_rank = jnp.dot(eq.astype(F32), ut_scr[...],
                      preferred_element_type=F32)
    sel = gt | (eq & (eq_rank < fill))
    rank = jnp.dot(sel.astype(F32), ut_scr[...],
                   preferred_element_type=F32)
    jrow = lax.broadcasted_iota(jnp.int32, (K, L), 0).astype(F32)
    lane = lax.broadcasted_iota(jnp.int32, (K, L), 1).astype(F32)
    for r in range(R):
        m = jnp.where((jrow == rank[r:r + 1, :]) & sel[r:r + 1, :], 1.0, 0.0)
        idx_ref[r] = jnp.sum(m * lane, axis=1).astype(jnp.int32)
        val_ref[r] = jnp.sum(m * v[r:r + 1, :], axis=1)


def _topk_call(vals2, K):
    R, L = vals2.shape
    return pl.pallas_call(
        functools.partial(_topk_body, R, L, K),
        grid=(1,),
        in_specs=[pl.BlockSpec((R, L), lambda i: (0, 0))],
        out_specs=[pl.BlockSpec((R, K), lambda i: (0, 0)),
                   pl.BlockSpec((R, K), lambda i: (0, 0))],
        out_shape=[jax.ShapeDtypeStruct((R, K), jnp.int32),
                   jax.ShapeDtypeStruct((R, K), F32)],
        scratch_shapes=[pltpu.VMEM((L, L), F32)],
    )(vals2)


NROWS = BW * G * (NQ + NKV)
NWORK = 32
PER_WORK = NROWS // NWORK
SC_CHUNK = 48
SC_NCH = PER_WORK // SC_CHUNK


def _sc_gather(xf, idx):
    mesh = plsc.VectorSubcoreMesh(core_axis_name="c", subcore_axis_name="s")

    @functools.partial(
        pl.kernel, mesh=mesh,
        out_type=jax.ShapeDtypeStruct((NROWS, DIM), F32),
        scratch_types=[
            pltpu.VMEM((PER_WORK,), jnp.int32),
            pltpu.VMEM((2, SC_CHUNK, DIM), F32),
            pltpu.SemaphoreType.DMA,
            pltpu.SemaphoreType.DMA,
            pltpu.SemaphoreType.DMA,
            pltpu.SemaphoreType.DMA,
        ],
    )
    def gk(idx_hbm, xf_hbm, out_hbm, idx_v, rows_v, g0, g1, w0, w1):
        wid = lax.axis_index("s") * 2 + lax.axis_index("c")
        base = wid * PER_WORK
        pltpu.sync_copy(idx_hbm.at[pl.ds(base, PER_WORK)], idx_v)
        gsem = (g0, g1)
        wsem = (w0, w1)
        gcp = [None, None]
        wcp = [None, None]
        gcp[0] = pltpu.async_copy(
            xf_hbm.at[idx_v.at[pl.ds(0, SC_CHUNK)]], rows_v.at[0], g0)
        for ci in range(SC_NCH):
            cur = ci % 2
            nxt = 1 - cur
            if ci + 1 < SC_NCH:
                if wcp[nxt] is not None:
                    wcp[nxt].wait()
                gcp[nxt] = pltpu.async_copy(
                    xf_hbm.at[idx_v.at[pl.ds((ci + 1) * SC_CHUNK, SC_CHUNK)]],
                    rows_v.at[nxt], gsem[nxt])
            gcp[cur].wait()
            wcp[cur] = pltpu.async_copy(
                rows_v.at[cur],
                out_hbm.at[pl.ds(base + ci * SC_CHUNK, SC_CHUNK)], wsem[cur])
        wcp[(SC_NCH - 1) % 2].wait()
        if SC_NCH >= 2:
            wcp[(SC_NCH - 2) % 2].wait()

    return gk(idx, xf)


def _scatter_body(af_ref, qif_ref, local_ref, out_ref):
    t = pl.program_id(0)
    routed = t % NB != 0
    loc = local_ref[0]
    qv = qif_ref[0, 0]
    pos = lax.broadcasted_iota(jnp.int32, (G * NQ, W), 1)
    P = (pos == qv[:, None]).astype(F32)
    af = af_ref[0]
    scat = lax.dot_general(P, af, (((0,), (0,)), ((), ())),
                           preferred_element_type=F32)
    cnt = jnp.sum(P, axis=0)
    add = scat / jnp.maximum(cnt, 1e-5)[:, None]
    out_ref[0] = loc + jnp.where(routed, add, 0.0)


def _scatter_call(af, qif, local3):
    def w_idx(t):
        return jnp.where(t % NB == 0, 0, t - t // NB - 1)

    return pl.pallas_call(
        _scatter_body,
        grid=(B * NB,),
        in_specs=[
            pl.BlockSpec((1, G * NQ, DIM), lambda t: (w_idx(t), 0, 0)),
            pl.BlockSpec((1, 1, G * NQ), lambda t: (w_idx(t), 0, 0)),
            pl.BlockSpec((1, W, DIM), lambda t: (t, 0, 0)),
        ],
        out_specs=pl.BlockSpec((1, W, DIM), lambda t: (t, 0, 0)),
        out_shape=jax.ShapeDtypeStruct((B * NB, W, DIM), F32),
    )(af, qif, local3)


def kernel(x, ln_g, ln_b, Wqkv, Wo_local, Wr_q, Wr_kv, gamma_q, gamma_c,
           Wq, Wkv, Wout, null_kv):
    x3 = x.reshape(B * NB, W, DIM)
    Wr_pad = jnp.concatenate(
        [Wr_q, Wr_kv, jnp.zeros((DIM, LOGPAD - 2 * G), F32)], axis=1)

    local3, logits = _fused_call(x3, ln_g.reshape(1, DIM), ln_b.reshape(1, DIM),
                                 Wqkv.astype(BF16), Wr_pad,
                                 Wo_local.astype(BF16))

    lq = logits[:, :, 0:G].reshape(B, NB, W, G)[:, 1:]
    lq = lq.transpose(0, 1, 3, 2).reshape(BW, G, W)
    q_idx3, q_vals3 = _topk_call(lq.reshape(BW * G, W), NQ)
    q_idx = q_idx3.reshape(BW, G, NQ)
    q_scores = jax.nn.sigmoid(q_vals3.reshape(BW, G, NQ))

    lkv = logits[:, :, G:2 * G].reshape(B, N, G)[:, :CTX]
    lkv = lkv.transpose(0, 2, 1)
    win = jnp.arange(NWIN)
    limits = (win + 1) * W
    ctx_mask = jnp.arange(CTX)[None, :] < limits[:, None]
    lkv_m = jnp.where(ctx_mask[None, :, None, :], lkv[:, None, :, :], -1e9)
    lkv_m = lkv_m.reshape(BW, G, CTX)
    kv_idx3, kv_vals3 = _topk_call(lkv_m.reshape(BW * G, CTX), NKV)
    kv_idx = kv_idx3.reshape(BW, G, NKV)
    kv_scores = jax.nn.sigmoid(kv_vals3.reshape(BW, G, NKV))
    lim_bw = limits[jnp.arange(BW) % NWIN]
    kv_valid = kv_idx < lim_bw[:, None, None]
    kv_bias = jnp.where(kv_valid, 0.0, -1e9).astype(F32)

    b_of = jnp.arange(BW) // NWIN
    w_of = jnp.arange(BW) % NWIN
    xf = x.reshape(B * N, DIM)
    q_glob = q_idx + ((w_of + 1) * W)[:, None, None] + (b_of * N)[:, None, None]
    kv_glob = kv_idx + (b_of * N)[:, None, None]
    all_idx = jnp.concatenate(
        [q_glob.reshape(-1), kv_glob.reshape(-1)]).astype(jnp.int32)
    gat = _sc_gather(xf, all_idx)

    nk = null_kv[0, :, :, 0, :]
    nv = null_kv[1, :, :, 0, :]
    attn_out = _grouped_call(
        gat,
        q_scores.reshape(BW * G, 1, NQ),
        kv_scores.reshape(BW * G, 1, NKV),
        kv_bias.reshape(BW * G, 1, NKV),
        gamma_q.reshape(G, 1, DIM), gamma_c.reshape(G, 1, DIM),
        Wq.astype(BF16), Wkv.astype(BF16), Wout.astype(BF16),
        nk, nv)

    af = attn_out.reshape(BW, G * NQ, DIM)
    qif = q_idx.reshape(BW, 1, G * NQ)
    out = _scatter_call(af, qif, local3)
    return out.reshape(B, N, DIM)
